# Initial kernel scaffold; baseline (speedup 1.0000x reference)
import jax
import jax.numpy as jnp
from jax.experimental import pallas as pl


def kernel(x, W, b):
    raise NotImplementedError("write your pallas kernel here")


# speedup vs baseline: 3.5877x; 3.5877x over previous
import functools

import jax
import jax.numpy as jnp
from jax.experimental import pallas as pl
from jax.experimental.pallas import tpu as pltpu

B, S, D, E, TOP_K = 4, 4096, 2048, 16, 2
T = 2048


def _router_body(x_ref, wt_ref, b_ref, out_ref):
    logits = jnp.dot(x_ref[...], wt_ref[...], preferred_element_type=jnp.float32)
    logits = logits + b_ref[...]
    m = jnp.max(logits, axis=-1, keepdims=True)
    e = jnp.exp(logits - m)
    p = e / jnp.sum(e, axis=-1, keepdims=True)
    lane = jax.lax.broadcasted_iota(jnp.int32, p.shape, 1)
    i1 = jnp.argmax(p, axis=-1, keepdims=True)
    p_masked = jnp.where(lane == i1, -jnp.inf, p)
    i2 = jnp.argmax(p_masked, axis=-1, keepdims=True)
    keep = (lane == i1) | (lane == i2)
    out_ref[...] = jnp.where(keep, p, 0.0)


@jax.jit
def kernel(x, W, b):
    xf = x.reshape(B * S, D)
    wt = W.T
    b2 = b.reshape(1, E)
    out = pl.pallas_call(
        _router_body,
        grid=(B * S // T,),
        in_specs=[
            pl.BlockSpec((T, D), lambda i: (i, 0)),
            pl.BlockSpec((D, E), lambda i: (0, 0)),
            pl.BlockSpec((1, E), lambda i: (0, 0)),
        ],
        out_specs=pl.BlockSpec((T, E), lambda i: (i, 0)),
        out_shape=jax.ShapeDtypeStruct((B * S, E), jnp.float32),
    )(xf, wt, b2)
    return out.reshape(B, S, E)
